# baseline (speedup 1.0000x reference)
import functools
import math

import jax
import jax.numpy as jnp
import numpy as np
from jax import lax
from jax.experimental import pallas as pl
from jax.experimental.pallas import tpu as pltpu
from jax.experimental.pallas import tpu_sc as plsc

C_IN = 256
H = 50
W = 50
NUM_ANCHORS = 9
STRIDE = 16
PRE_NMS_TOP_N = 6000
POST_NMS_TOP_N = 1000
NMS_THRESH = 0.7
MIN_SIZE = 1.0
BBOX_XFORM_CLIP = math.log(1000.0 / 16)

GRID = 52
NFLAT = GRID * GRID
NPAD = 2816
XEXT = 2944
NSEL = 25600
SROWS = 200
NEG_INF = float("-inf")
BIG_I32 = np.int32(1 << 30)


def _build_consts():
    sizes = np.array([128.0, 256.0, 512.0])
    ratios = np.array([0.5, 1.0, 2.0])
    hs, ws = [], []
    for s in sizes:
        for r in ratios:
            hs.append(s * np.sqrt(r))
            ws.append(s / np.sqrt(r))
    hs = np.array(hs, np.float64)
    ws = np.array(ws, np.float64)

    hh = np.arange(GRID)[:, None].repeat(GRID, 1)
    ww = np.arange(GRID)[None, :].repeat(GRID, 0)
    valid = (hh >= 1) & (hh <= H) & (ww >= 1) & (ww <= W)
    h = hh - 1
    w = ww - 1
    cx = (w + 0.5) * STRIDE
    cy = (h + 0.5) * STRIDE

    def flat_pad(a2d, fill):
        flat = a2d.reshape(-1)
        out = np.full((NPAD,), fill, a2d.dtype)
        out[:NFLAT] = flat
        return out

    cxf = flat_pad(cx.astype(np.float32), 0.0)
    cyf = flat_pad(cy.astype(np.float32), 0.0)
    validf = flat_pad(valid, False)

    CW = np.broadcast_to(ws.astype(np.float32)[:, None], (NUM_ANCHORS, NPAD)).copy()
    CH = np.broadcast_to(hs.astype(np.float32)[:, None], (NUM_ANCHORS, NPAD)).copy()
    CX = np.broadcast_to(cxf[None, :], (NUM_ANCHORS, NPAD)).copy()
    CY = np.broadcast_to(cyf[None, :], (NUM_ANCHORS, NPAD)).copy()

    posmask = np.where(validf[None, :], 0.0, NEG_INF).astype(np.float32)
    posmask = np.broadcast_to(posmask, (NUM_ANCHORS, NPAD)).copy()

    hwf = flat_pad((np.minimum(h, H - 1) * W + np.minimum(w, W - 1)).astype(np.int64), 0)
    refidx = hwf[None, :] * NUM_ANCHORS + np.arange(NUM_ANCHORS)[:, None]
    refidx = np.where(np.broadcast_to(validf[None, :], refidx.shape), refidx, BIG_I32)
    refidx = refidx.astype(np.int32)
    return CW, CH, CX, CY, posmask, refidx


_CW, _CH, _CX, _CY, _POSMASK, _REFIDX = _build_consts()
_TAP_OFFS = tuple(kh * GRID + kw for kh in range(3) for kw in range(3))


def _dense_body(x_ext_ref, wconv_ref, bconv_ref, whead_ref, bhead_ref,
                cw_ref, ch_ref, cx_ref, cy_ref, posmask_ref, img_ref,
                sraw_ref, snms_ref, x1_ref, y1_ref, x2_ref, y2_ref):
    acc = jnp.zeros((C_IN, NPAD), jnp.float32)
    for k, off in enumerate(_TAP_OFFS):
        acc += jnp.dot(wconv_ref[k], x_ext_ref[:, off:off + NPAD],
                       preferred_element_type=jnp.float32)
    act = jnp.maximum(acc + bconv_ref[:, 0:1], 0.0)
    heads = jnp.dot(whead_ref[...], act, preferred_element_type=jnp.float32)
    heads = heads + bhead_ref[:, 0:1]

    logits = heads[0:9]
    dx = heads[9:18]
    dy = heads[18:27]
    dw = jnp.minimum(heads[27:36], BBOX_XFORM_CLIP)
    dh = jnp.minimum(heads[36:45], BBOX_XFORM_CLIP)

    cw = cw_ref[...]
    ch = ch_ref[...]
    pcx = dx * cw + cx_ref[...]
    pcy = dy * ch + cy_ref[...]
    pw = jnp.exp(dw) * cw
    ph = jnp.exp(dh) * ch

    img = img_ref[0, 0]
    x1 = jnp.clip(pcx - 0.5 * pw, 0.0, img)
    y1 = jnp.clip(pcy - 0.5 * ph, 0.0, img)
    x2 = jnp.clip(pcx + 0.5 * pw, 0.0, img)
    y2 = jnp.clip(pcy + 0.5 * ph, 0.0, img)

    sraw = logits + posmask_ref[...]
    valid = ((x2 - x1) >= MIN_SIZE) & ((y2 - y1) >= MIN_SIZE)
    snms = jnp.where(valid, sraw, NEG_INF)

    sraw_ref[...] = sraw
    snms_ref[...] = snms
    x1_ref[...] = x1
    y1_ref[...] = y1
    x2_ref[...] = x2
    y2_ref[...] = y2


def _count_ge(keys, cand):
    return jnp.sum((keys >= cand).astype(jnp.int32))


def _sel_body(sraw_ref, snms_ref, x1_ref, y1_ref, x2_ref, y2_ref,
              refidx_ref, ehot_ref, s0_ref, fb_ref):
    sraw = sraw_ref[...]
    bits = jax.lax.bitcast_convert_type(sraw, jnp.int32)
    keys = jnp.where(bits < 0, bits ^ jnp.int32(0x7FFFFFFF), bits)

    cpos = _count_ge(keys, jnp.int32(0))
    k_val = jnp.where(cpos >= PRE_NMS_TOP_N, jnp.int32(0), jnp.int32(-2147483648))
    for bit in range(30, -1, -1):
        cand = k_val | jnp.int32(1 << bit)
        k_val = jnp.where(_count_ge(keys, cand) >= PRE_NMS_TOP_N, cand, k_val)

    c_gt = jnp.sum((keys > k_val).astype(jnp.int32))
    m_ties = PRE_NMS_TOP_N - c_gt
    tie = keys == k_val
    refidx = refidx_ref[...]
    lo = jnp.int32(0)
    hi = jnp.int32((1 << 15) - 1)
    for _ in range(15):
        mid = (lo + hi) // 2
        cnt = jnp.sum((tie & (refidx <= mid)).astype(jnp.int32))
        take = cnt >= m_ties
        hi = jnp.where(take, mid, hi)
        lo = jnp.where(take, lo, mid + 1)
    in_topk = (keys > k_val) | (tie & (refidx <= hi))

    s0_ref[...] = jnp.where(in_topk, snms_ref[...], NEG_INF)

    m0 = jnp.max(sraw)
    i0 = jnp.min(jnp.where(sraw == m0, refidx, BIG_I32))
    ch0 = ((sraw == m0) & (refidx == i0)).astype(jnp.float32)
    fb0 = jnp.sum(ch0 * x1_ref[...])
    fb1 = jnp.sum(ch0 * y1_ref[...])
    fb2 = jnp.sum(ch0 * x2_ref[...])
    fb3 = jnp.sum(ch0 * y2_ref[...])
    fb_ref[0:1, :] = (fb0 * ehot_ref[0:1, :] + fb1 * ehot_ref[1:2, :]
                      + fb2 * ehot_ref[2:3, :] + fb3 * ehot_ref[3:4, :])


SC_NT = 16
SC_CHUNK = 1600
SC_NBLK = SC_CHUNK // 16
SC_PAD = 1664
SC_SEG = 416
NCMP = 6656
SC_OUT = 6784
SC_DUMP = NCMP


def _sc_compact_body(s0_hbm, x1_hbm, y1_hbm, x2_hbm, y2_hbm,
                     os_hbm, ox1_hbm, oy1_hbm, ox2_hbm, oy2_hbm, cnt_hbm,
                     sv, xv1, yv1, xv2, yv2, idx1d, idx2d,
                     cls_v, clc_v, cntb, ctv, tmpf, sem):
    w = lax.axis_index("s")
    base_in = w * SC_CHUNK

    pltpu.sync_copy(s0_hbm.at[pl.ds(base_in, SC_CHUNK)], sv.at[pl.ds(0, SC_CHUNK)])
    pltpu.sync_copy(x1_hbm.at[pl.ds(base_in, SC_CHUNK)], xv1.at[pl.ds(0, SC_CHUNK)])
    pltpu.sync_copy(y1_hbm.at[pl.ds(base_in, SC_CHUNK)], yv1.at[pl.ds(0, SC_CHUNK)])
    pltpu.sync_copy(x2_hbm.at[pl.ds(base_in, SC_CHUNK)], xv2.at[pl.ds(0, SC_CHUNK)])
    pltpu.sync_copy(y2_hbm.at[pl.ds(base_in, SC_CHUNK)], yv2.at[pl.ds(0, SC_CHUNK)])

    for j in range(SC_SEG // 16):
        cls_v[pl.ds(j * 16, 16)] = jnp.full((16,), NEG_INF, jnp.float32)
        clc_v[pl.ds(j * 16, 16)] = jnp.zeros((16,), jnp.float32)
    seg = w * SC_SEG
    pltpu.sync_copy(cls_v, os_hbm.at[pl.ds(seg, SC_SEG)])
    pltpu.sync_copy(clc_v, ox1_hbm.at[pl.ds(seg, SC_SEG)])
    pltpu.sync_copy(clc_v, oy1_hbm.at[pl.ds(seg, SC_SEG)])
    pltpu.sync_copy(clc_v, ox2_hbm.at[pl.ds(seg, SC_SEG)])
    pltpu.sync_copy(clc_v, oy2_hbm.at[pl.ds(seg, SC_SEG)])

    ones = jnp.full((16,), 1, jnp.int32)
    neg1 = jnp.full((16,), -1, jnp.int32)
    zeros16 = ones - ones
    ninf = jnp.full((16,), NEG_INF, jnp.float32)
    onef = jnp.full((16,), 1.0, jnp.float32)
    zerof = jnp.full((16,), 0.0, jnp.float32)
    lane = lax.iota(jnp.int32, 16)

    tmpf[pl.ds(0, 16)] = zerof
    tmpf[pl.ds(32, 16)] = zerof

    def prefix16(vf):
        acc = vf
        for st in (1, 2, 4, 8):
            tmpf[pl.ds(16, 16)] = acc
            acc = acc + tmpf[pl.ds(16 - st, 16)]
        return acc

    def splat_last(nondec):
        m = nondec
        for st in (1, 2, 4, 8):
            tmpf[pl.ds(16, 16)] = m
            m = jnp.maximum(m, tmpf[pl.ds(16 + st, 16)])
        return m

    run = jnp.full((16,), 0, jnp.int32)
    for j in range(SC_NBLK):
        v = sv[pl.ds(j * 16, 16)]
        m = v > ninf
        mif = jnp.where(m, onef, zerof)
        pcf = prefix16(mif)
        pc = pcf.astype(jnp.int32)
        idx1d[pl.ds(j * 16, 16)] = jnp.where(m, run + pc - ones, neg1)
        run = run + splat_last(pcf).astype(jnp.int32)
    for j in range(SC_NBLK, SC_PAD // 16):
        idx1d[pl.ds(j * 16, 16)] = neg1

    ctv[...] = run
    pltpu.sync_copy(ctv, cnt_hbm.at[w])
    plsc.subcore_barrier()
    pltpu.sync_copy(cnt_hbm, cntb)
    counts = zeros16
    for j in range(SC_NT):
        jv = jnp.full((16,), j, jnp.int32)
        counts = jnp.where(lane == jv, cntb[j], counts)
    wv = jnp.full((16,), w, jnp.int32)
    cmask = jnp.where(lane < wv, counts.astype(jnp.float32), zerof)
    base = splat_last(prefix16(cmask)).astype(jnp.int32)

    dump = jnp.full((16,), SC_DUMP, jnp.int32)
    for j in range(SC_PAD // 16):
        dv = idx1d[pl.ds(j * 16, 16)]
        dv = jnp.where(dv >= zeros16, dv + base, dump)
        idx2d[j // 8, pl.ds((j % 8) * 16, 16)] = dv

    for data, outref in ((sv, os_hbm), (xv1, ox1_hbm), (yv1, oy1_hbm),
                         (xv2, ox2_hbm), (yv2, oy2_hbm)):
        handles = [
            pltpu.async_copy(data.at[pl.ds(r * 128, 128)],
                             outref.at[idx2d.at[r]], sem)
            for r in range(SC_PAD // 128)
        ]
        for h in handles:
            h.wait()


def _sc_compact(s0_flat, x1f, y1f, x2f, y2f):
    f32 = jnp.float32
    i32 = jnp.int32
    mesh = plsc.VectorSubcoreMesh(core_axis_name="c", subcore_axis_name="s",
                                  num_cores=1)
    out_type = [jax.ShapeDtypeStruct((SC_OUT,), f32)] * 5 + [
        jax.ShapeDtypeStruct((SC_NT, 16), i32)]
    scratch = [
        pltpu.VMEM((SC_PAD,), f32),
        pltpu.VMEM((SC_PAD,), f32),
        pltpu.VMEM((SC_PAD,), f32),
        pltpu.VMEM((SC_PAD,), f32),
        pltpu.VMEM((SC_PAD,), f32),
        pltpu.VMEM((SC_PAD,), i32),
        pltpu.VMEM((SC_PAD // 128, 128), i32),
        pltpu.VMEM((SC_SEG,), f32),
        pltpu.VMEM((SC_SEG,), f32),
        pltpu.VMEM((SC_NT, 16), i32),
        pltpu.VMEM((16,), i32),
        pltpu.VMEM((48,), f32),
        pltpu.SemaphoreType.DMA,
    ]
    fn = pl.kernel(_sc_compact_body, mesh=mesh, out_type=out_type,
                   scratch_types=scratch)
    outs = fn(s0_flat, x1f, y1f, x2f, y2f)
    return outs[:5]


def _nms_body(sc_ref, x1_ref, y1_ref, x2_ref, y2_ref, fb_ref, ehot_ref,
              out_ref):
    s0 = sc_ref[...]
    x1 = x1_ref[...]
    y1 = y1_ref[...]
    x2 = x2_ref[...]
    y2 = y2_ref[...]
    areas = (x2 - x1) * (y2 - y1)

    e0 = ehot_ref[0:1, :]
    e1 = ehot_ref[1:2, :]
    e2 = ehot_ref[2:3, :]
    e3 = ehot_ref[3:4, :]
    fbrow = fb_ref[0:1, :]
    fb0 = jnp.sum(fbrow * e0)
    fb1 = jnp.sum(fbrow * e1)
    fb2 = jnp.sum(fbrow * e2)
    fb3 = jnp.sum(fbrow * e3)

    def body(it, s):
        mval = jnp.max(s)
        cf = (s == mval).astype(jnp.float32)
        isfb = mval == NEG_INF
        b0 = jnp.where(isfb, fb0, jnp.sum(cf * x1))
        b1 = jnp.where(isfb, fb1, jnp.sum(cf * y1))
        b2 = jnp.where(isfb, fb2, jnp.sum(cf * x2))
        b3 = jnp.where(isfb, fb3, jnp.sum(cf * y2))
        ar = (b2 - b0) * (b3 - b1)

        xx1 = jnp.maximum(b0, x1)
        yy1 = jnp.maximum(b1, y1)
        xx2 = jnp.minimum(b2, x2)
        yy2 = jnp.minimum(b3, y2)
        inter = jnp.maximum(xx2 - xx1, 0.0) * jnp.maximum(yy2 - yy1, 0.0)
        iou = inter / (ar + areas - inter + 1e-9)
        s = jnp.where(iou > NMS_THRESH, NEG_INF, s)

        out_ref[pl.ds(it, 1), :] = b0 * e0 + b1 * e1 + b2 * e2 + b3 * e3
        return s

    jax.lax.fori_loop(0, POST_NMS_TOP_N, body, s0)


@functools.partial(jax.jit, static_argnames=())
def kernel(features, conv_w, conv_b, cls_w, cls_b, reg_w, reg_b, image_shape):
    f32 = jnp.float32

    x = features[0].astype(f32)
    x_pad = jnp.pad(x, ((0, 0), (1, 1), (1, 1)))
    x_flat = x_pad.reshape(C_IN, NFLAT)
    x_ext = jnp.pad(x_flat, ((0, 0), (53, XEXT - NFLAT - 53)))

    wconv = jnp.transpose(conv_w, (2, 3, 0, 1)).reshape(9, C_IN, C_IN)
    bconv = conv_b.reshape(C_IN, 1)

    wcls = cls_w.reshape(NUM_ANCHORS, C_IN)
    wreg = reg_w.reshape(NUM_ANCHORS, 4, C_IN).transpose(1, 0, 2).reshape(36, C_IN)
    whead = jnp.concatenate([wcls, wreg, jnp.zeros((3, C_IN), f32)], axis=0)
    bcls = cls_b.reshape(NUM_ANCHORS)
    breg = reg_b.reshape(NUM_ANCHORS, 4).transpose(1, 0).reshape(36)
    bhead = jnp.concatenate([bcls, breg, jnp.zeros((3,), f32)]).reshape(48, 1)

    img = jnp.asarray(image_shape, f32).reshape(1, 1)

    shp = (NUM_ANCHORS, NPAD)
    dense_out = pl.pallas_call(
        _dense_body,
        out_shape=[jax.ShapeDtypeStruct(shp, f32)] * 6,
        in_specs=[
            pl.BlockSpec(memory_space=pltpu.VMEM),
            pl.BlockSpec(memory_space=pltpu.VMEM),
            pl.BlockSpec(memory_space=pltpu.VMEM),
            pl.BlockSpec(memory_space=pltpu.VMEM),
            pl.BlockSpec(memory_space=pltpu.VMEM),
            pl.BlockSpec(memory_space=pltpu.VMEM),
            pl.BlockSpec(memory_space=pltpu.VMEM),
            pl.BlockSpec(memory_space=pltpu.VMEM),
            pl.BlockSpec(memory_space=pltpu.VMEM),
            pl.BlockSpec(memory_space=pltpu.VMEM),
            pl.BlockSpec(memory_space=pltpu.SMEM),
        ],
        out_specs=[pl.BlockSpec(memory_space=pltpu.VMEM)] * 6,
    )(x_ext, wconv, bconv, whead, bhead,
      jnp.asarray(_CW), jnp.asarray(_CH), jnp.asarray(_CX), jnp.asarray(_CY),
      jnp.asarray(_POSMASK), img)

    sraw, snms, x1, y1, x2, y2 = dense_out

    def to_sel(a, fill):
        flat = a.reshape(-1)
        return jnp.pad(flat, (0, NSEL - flat.shape[0]),
                       constant_values=fill).reshape(SROWS, 128)

    sraw_s = to_sel(sraw, NEG_INF)
    snms_s = to_sel(snms, NEG_INF)
    x1_s = to_sel(x1, 0.0)
    y1_s = to_sel(y1, 0.0)
    x2_s = to_sel(x2, 0.0)
    y2_s = to_sel(y2, 0.0)

    refidx_s = jnp.pad(jnp.asarray(_REFIDX).reshape(-1),
                       (0, NSEL - NUM_ANCHORS * NPAD),
                       constant_values=BIG_I32).reshape(SROWS, 128)
    ehot = jnp.zeros((8, 128), f32).at[jnp.arange(4), jnp.arange(4)].set(1.0)

    s0_s, fbrow = pl.pallas_call(
        _sel_body,
        out_shape=[jax.ShapeDtypeStruct((SROWS, 128), f32),
                   jax.ShapeDtypeStruct((8, 128), f32)],
        in_specs=[pl.BlockSpec(memory_space=pltpu.VMEM)] * 8,
        out_specs=[pl.BlockSpec(memory_space=pltpu.VMEM)] * 2,
    )(sraw_s, snms_s, x1_s, y1_s, x2_s, y2_s, refidx_s, ehot)

    cs, cx1, cy1, cx2, cy2 = _sc_compact(
        s0_s.reshape(-1), x1_s.reshape(-1), y1_s.reshape(-1),
        x2_s.reshape(-1), y2_s.reshape(-1))

    def to_cmp(a):
        return a[:NCMP].reshape(NCMP // 128, 128)

    out = pl.pallas_call(
        _nms_body,
        out_shape=jax.ShapeDtypeStruct((1024, 128), f32),
        in_specs=[pl.BlockSpec(memory_space=pltpu.VMEM)] * 7,
        out_specs=pl.BlockSpec(memory_space=pltpu.VMEM),
    )(to_cmp(cs), to_cmp(cx1), to_cmp(cy1), to_cmp(cx2), to_cmp(cy2),
      fbrow, ehot)

    return out[:POST_NMS_TOP_N, :4]


# speedup vs baseline: 5.9023x; 5.9023x over previous
import functools
import math

import jax
import jax.numpy as jnp
import numpy as np
from jax import lax
from jax.experimental import pallas as pl
from jax.experimental.pallas import tpu as pltpu
from jax.experimental.pallas import tpu_sc as plsc

C_IN = 256
H = 50
W = 50
NUM_ANCHORS = 9
STRIDE = 16
PRE_NMS_TOP_N = 6000
POST_NMS_TOP_N = 1000
NMS_THRESH = 0.7
MIN_SIZE = 1.0
BBOX_XFORM_CLIP = math.log(1000.0 / 16)

GRID = 52
NFLAT = GRID * GRID
NPAD = 2816
XEXT = 2944
NSEL = 25600
SROWS = 200
NEG_INF = float("-inf")
BIG_I32 = np.int32(1 << 30)


def _build_consts():
    sizes = np.array([128.0, 256.0, 512.0])
    ratios = np.array([0.5, 1.0, 2.0])
    hs, ws = [], []
    for s in sizes:
        for r in ratios:
            hs.append(s * np.sqrt(r))
            ws.append(s / np.sqrt(r))
    hs = np.array(hs, np.float64)
    ws = np.array(ws, np.float64)

    hh = np.arange(GRID)[:, None].repeat(GRID, 1)
    ww = np.arange(GRID)[None, :].repeat(GRID, 0)
    valid = (hh >= 1) & (hh <= H) & (ww >= 1) & (ww <= W)
    h = hh - 1
    w = ww - 1
    cx = (w + 0.5) * STRIDE
    cy = (h + 0.5) * STRIDE

    def flat_pad(a2d, fill):
        flat = a2d.reshape(-1)
        out = np.full((NPAD,), fill, a2d.dtype)
        out[:NFLAT] = flat
        return out

    cxf = flat_pad(cx.astype(np.float32), 0.0)
    cyf = flat_pad(cy.astype(np.float32), 0.0)
    validf = flat_pad(valid, False)

    CW = np.broadcast_to(ws.astype(np.float32)[:, None], (NUM_ANCHORS, NPAD)).copy()
    CH = np.broadcast_to(hs.astype(np.float32)[:, None], (NUM_ANCHORS, NPAD)).copy()
    CX = np.broadcast_to(cxf[None, :], (NUM_ANCHORS, NPAD)).copy()
    CY = np.broadcast_to(cyf[None, :], (NUM_ANCHORS, NPAD)).copy()

    posmask = np.where(validf[None, :], 0.0, NEG_INF).astype(np.float32)
    posmask = np.broadcast_to(posmask, (NUM_ANCHORS, NPAD)).copy()

    hwf = flat_pad((np.minimum(h, H - 1) * W + np.minimum(w, W - 1)).astype(np.int64), 0)
    refidx = hwf[None, :] * NUM_ANCHORS + np.arange(NUM_ANCHORS)[:, None]
    refidx = np.where(np.broadcast_to(validf[None, :], refidx.shape), refidx, BIG_I32)
    refidx = refidx.astype(np.int32)
    return CW, CH, CX, CY, posmask, refidx


_CW, _CH, _CX, _CY, _POSMASK, _REFIDX = _build_consts()
_TAP_OFFS = tuple(kh * GRID + kw for kh in range(3) for kw in range(3))


def _dense_body(x_ext_ref, wconv_ref, bconv_ref, whead_ref, bhead_ref,
                cw_ref, ch_ref, cx_ref, cy_ref, posmask_ref, img_ref,
                sraw_ref, snms_ref, x1_ref, y1_ref, x2_ref, y2_ref):
    acc = jnp.zeros((C_IN, NPAD), jnp.float32)
    for k, off in enumerate(_TAP_OFFS):
        acc += jnp.dot(wconv_ref[k], x_ext_ref[:, off:off + NPAD],
                       preferred_element_type=jnp.float32)
    act = jnp.maximum(acc + bconv_ref[:, 0:1], 0.0)
    heads = jnp.dot(whead_ref[...], act, preferred_element_type=jnp.float32)
    heads = heads + bhead_ref[:, 0:1]

    logits = heads[0:9]
    dx = heads[9:18]
    dy = heads[18:27]
    dw = jnp.minimum(heads[27:36], BBOX_XFORM_CLIP)
    dh = jnp.minimum(heads[36:45], BBOX_XFORM_CLIP)

    cw = cw_ref[...]
    ch = ch_ref[...]
    pcx = dx * cw + cx_ref[...]
    pcy = dy * ch + cy_ref[...]
    pw = jnp.exp(dw) * cw
    ph = jnp.exp(dh) * ch

    img = img_ref[0, 0]
    x1 = jnp.clip(pcx - 0.5 * pw, 0.0, img)
    y1 = jnp.clip(pcy - 0.5 * ph, 0.0, img)
    x2 = jnp.clip(pcx + 0.5 * pw, 0.0, img)
    y2 = jnp.clip(pcy + 0.5 * ph, 0.0, img)

    sraw = logits + posmask_ref[...]
    valid = ((x2 - x1) >= MIN_SIZE) & ((y2 - y1) >= MIN_SIZE)
    snms = jnp.where(valid, sraw, NEG_INF)

    sraw_ref[...] = sraw
    snms_ref[...] = snms
    x1_ref[...] = x1
    y1_ref[...] = y1
    x2_ref[...] = x2
    y2_ref[...] = y2


def _count_ge(keys, cand):
    return jnp.sum((keys >= cand).astype(jnp.int32))


def _sel_body(sraw_ref, snms_ref, x1_ref, y1_ref, x2_ref, y2_ref,
              refidx_ref, ehot_ref, s0_ref, fb_ref):
    sraw = sraw_ref[...]
    bits = jax.lax.bitcast_convert_type(sraw, jnp.int32)
    keys = jnp.where(bits < 0, bits ^ jnp.int32(0x7FFFFFFF), bits)

    cpos = _count_ge(keys, jnp.int32(0))
    k_val = jnp.where(cpos >= PRE_NMS_TOP_N, jnp.int32(0), jnp.int32(-2147483648))
    for bit in range(30, -1, -1):
        cand = k_val | jnp.int32(1 << bit)
        k_val = jnp.where(_count_ge(keys, cand) >= PRE_NMS_TOP_N, cand, k_val)

    c_gt = jnp.sum((keys > k_val).astype(jnp.int32))
    m_ties = PRE_NMS_TOP_N - c_gt
    tie = keys == k_val
    refidx = refidx_ref[...]
    lo = jnp.int32(0)
    hi = jnp.int32((1 << 15) - 1)
    for _ in range(15):
        mid = (lo + hi) // 2
        cnt = jnp.sum((tie & (refidx <= mid)).astype(jnp.int32))
        take = cnt >= m_ties
        hi = jnp.where(take, mid, hi)
        lo = jnp.where(take, lo, mid + 1)
    in_topk = (keys > k_val) | (tie & (refidx <= hi))

    s0_ref[...] = jnp.where(in_topk, snms_ref[...], NEG_INF)

    m0 = jnp.max(sraw)
    i0 = jnp.min(jnp.where(sraw == m0, refidx, BIG_I32))
    ch0 = ((sraw == m0) & (refidx == i0)).astype(jnp.float32)
    fb0 = jnp.sum(ch0 * x1_ref[...])
    fb1 = jnp.sum(ch0 * y1_ref[...])
    fb2 = jnp.sum(ch0 * x2_ref[...])
    fb3 = jnp.sum(ch0 * y2_ref[...])
    fb_ref[0:1, :] = (fb0 * ehot_ref[0:1, :] + fb1 * ehot_ref[1:2, :]
                      + fb2 * ehot_ref[2:3, :] + fb3 * ehot_ref[3:4, :])


SC_NT = 16
SC_CHUNK = 1600
SC_NBLK = SC_CHUNK // 16
SC_PAD = 1664
SC_SEG = 416
NCMP = 6656
SC_OUT = 8704
SC_DUMP = NCMP


def _sc_compact_body(s0_hbm, x1_hbm, y1_hbm, x2_hbm, y2_hbm,
                     os_hbm, ox1_hbm, oy1_hbm, ox2_hbm, oy2_hbm, cnt_hbm,
                     sv, xv1, yv1, xv2, yv2, idx1d, idx2d,
                     cls_v, clc_v, cntb, ctv, tmpf, sem):
    w = lax.axis_index("s")
    base_in = w * SC_CHUNK

    pltpu.sync_copy(s0_hbm.at[pl.ds(base_in, SC_CHUNK)], sv.at[pl.ds(0, SC_CHUNK)])
    pltpu.sync_copy(x1_hbm.at[pl.ds(base_in, SC_CHUNK)], xv1.at[pl.ds(0, SC_CHUNK)])
    pltpu.sync_copy(y1_hbm.at[pl.ds(base_in, SC_CHUNK)], yv1.at[pl.ds(0, SC_CHUNK)])
    pltpu.sync_copy(x2_hbm.at[pl.ds(base_in, SC_CHUNK)], xv2.at[pl.ds(0, SC_CHUNK)])
    pltpu.sync_copy(y2_hbm.at[pl.ds(base_in, SC_CHUNK)], yv2.at[pl.ds(0, SC_CHUNK)])

    for j in range(SC_SEG // 16):
        cls_v[pl.ds(j * 16, 16)] = jnp.full((16,), NEG_INF, jnp.float32)
        clc_v[pl.ds(j * 16, 16)] = jnp.zeros((16,), jnp.float32)
    seg = w * SC_SEG
    pltpu.sync_copy(cls_v, os_hbm.at[pl.ds(seg, SC_SEG)])
    pltpu.sync_copy(clc_v, ox1_hbm.at[pl.ds(seg, SC_SEG)])
    pltpu.sync_copy(clc_v, oy1_hbm.at[pl.ds(seg, SC_SEG)])
    pltpu.sync_copy(clc_v, ox2_hbm.at[pl.ds(seg, SC_SEG)])
    pltpu.sync_copy(clc_v, oy2_hbm.at[pl.ds(seg, SC_SEG)])

    ones = jnp.full((16,), 1, jnp.int32)
    neg1 = jnp.full((16,), -1, jnp.int32)
    zeros16 = ones - ones
    ninf = jnp.full((16,), NEG_INF, jnp.float32)
    onef = jnp.full((16,), 1.0, jnp.float32)
    zerof = jnp.full((16,), 0.0, jnp.float32)
    lane = lax.iota(jnp.int32, 16)

    tmpf[pl.ds(0, 16)] = zerof
    tmpf[pl.ds(32, 16)] = zerof

    def prefix16(vf):
        acc = vf
        for st in (1, 2, 4, 8):
            tmpf[pl.ds(16, 16)] = acc
            acc = acc + tmpf[pl.ds(16 - st, 16)]
        return acc

    def splat_last(nondec):
        m = nondec
        for st in (1, 2, 4, 8):
            tmpf[pl.ds(16, 16)] = m
            m = jnp.maximum(m, tmpf[pl.ds(16 + st, 16)])
        return m

    run = jnp.full((16,), 0, jnp.int32)
    for j in range(SC_NBLK):
        v = sv[pl.ds(j * 16, 16)]
        m = v > ninf
        mif = jnp.where(m, onef, zerof)
        pcf = prefix16(mif)
        pc = pcf.astype(jnp.int32)
        idx1d[pl.ds(j * 16, 16)] = jnp.where(m, run + pc - ones, neg1)
        run = run + splat_last(pcf).astype(jnp.int32)
    for j in range(SC_NBLK, SC_PAD // 16):
        idx1d[pl.ds(j * 16, 16)] = neg1

    ctv[...] = run
    pltpu.sync_copy(ctv, cnt_hbm.at[w])
    plsc.subcore_barrier()
    pltpu.sync_copy(cnt_hbm, cntb)
    counts = zeros16
    for j in range(SC_NT):
        jv = jnp.full((16,), j, jnp.int32)
        counts = jnp.where(lane == jv, cntb[j], counts)
    wv = jnp.full((16,), w, jnp.int32)
    cmask = jnp.where(lane < wv, counts.astype(jnp.float32), zerof)
    base = splat_last(prefix16(cmask)).astype(jnp.int32)

    dumpbase = jnp.full((16,), SC_DUMP, jnp.int32) + wv * jnp.full(
        (16,), 128, jnp.int32) + lane
    for j in range(SC_PAD // 16):
        dv = idx1d[pl.ds(j * 16, 16)]
        dump = dumpbase + jnp.full((16,), (j % 8) * 16, jnp.int32)
        dv = jnp.where(dv >= zeros16, dv + base, dump)
        idx2d[j // 8, pl.ds((j % 8) * 16, 16)] = dv

    for data, outref in ((sv, os_hbm), (xv1, ox1_hbm), (yv1, oy1_hbm),
                         (xv2, ox2_hbm), (yv2, oy2_hbm)):
        handles = [
            pltpu.async_copy(data.at[pl.ds(r * 128, 128)],
                             outref.at[idx2d.at[r]], sem)
            for r in range(SC_PAD // 128)
        ]
        for h in handles:
            h.wait()


def _sc_compact(s0_flat, x1f, y1f, x2f, y2f):
    f32 = jnp.float32
    i32 = jnp.int32
    mesh = plsc.VectorSubcoreMesh(core_axis_name="c", subcore_axis_name="s",
                                  num_cores=1)
    out_type = [jax.ShapeDtypeStruct((SC_OUT,), f32)] * 5 + [
        jax.ShapeDtypeStruct((SC_NT, 16), i32)]
    scratch = [
        pltpu.VMEM((SC_PAD,), f32),
        pltpu.VMEM((SC_PAD,), f32),
        pltpu.VMEM((SC_PAD,), f32),
        pltpu.VMEM((SC_PAD,), f32),
        pltpu.VMEM((SC_PAD,), f32),
        pltpu.VMEM((SC_PAD,), i32),
        pltpu.VMEM((SC_PAD // 128, 128), i32),
        pltpu.VMEM((SC_SEG,), f32),
        pltpu.VMEM((SC_SEG,), f32),
        pltpu.VMEM((SC_NT, 16), i32),
        pltpu.VMEM((16,), i32),
        pltpu.VMEM((48,), f32),
        pltpu.SemaphoreType.DMA,
    ]
    fn = pl.kernel(_sc_compact_body, mesh=mesh, out_type=out_type,
                   scratch_types=scratch)
    outs = fn(s0_flat, x1f, y1f, x2f, y2f)
    return outs[:5]


def _nms_body(sc_ref, x1_ref, y1_ref, x2_ref, y2_ref, fb_ref, ehot_ref,
              out_ref):
    s0 = sc_ref[...]
    x1 = x1_ref[...]
    y1 = y1_ref[...]
    x2 = x2_ref[...]
    y2 = y2_ref[...]
    areas = (x2 - x1) * (y2 - y1)

    e0 = ehot_ref[0:1, :]
    e1 = ehot_ref[1:2, :]
    e2 = ehot_ref[2:3, :]
    e3 = ehot_ref[3:4, :]
    fbrow = fb_ref[0:1, :]
    fb0 = jnp.sum(fbrow * e0)
    fb1 = jnp.sum(fbrow * e1)
    fb2 = jnp.sum(fbrow * e2)
    fb3 = jnp.sum(fbrow * e3)

    def body(it, s):
        mval = jnp.max(s)
        cf = (s == mval).astype(jnp.float32)
        isfb = mval == NEG_INF
        b0 = jnp.where(isfb, fb0, jnp.sum(cf * x1))
        b1 = jnp.where(isfb, fb1, jnp.sum(cf * y1))
        b2 = jnp.where(isfb, fb2, jnp.sum(cf * x2))
        b3 = jnp.where(isfb, fb3, jnp.sum(cf * y2))
        ar = (b2 - b0) * (b3 - b1)

        xx1 = jnp.maximum(b0, x1)
        yy1 = jnp.maximum(b1, y1)
        xx2 = jnp.minimum(b2, x2)
        yy2 = jnp.minimum(b3, y2)
        inter = jnp.maximum(xx2 - xx1, 0.0) * jnp.maximum(yy2 - yy1, 0.0)
        iou = inter / (ar + areas - inter + 1e-9)
        s = jnp.where(iou > NMS_THRESH, NEG_INF, s)

        out_ref[pl.ds(it, 1), :] = b0 * e0 + b1 * e1 + b2 * e2 + b3 * e3
        return s

    jax.lax.fori_loop(0, POST_NMS_TOP_N, body, s0)


@functools.partial(jax.jit, static_argnames=())
def kernel(features, conv_w, conv_b, cls_w, cls_b, reg_w, reg_b, image_shape):
    f32 = jnp.float32

    x = features[0].astype(f32)
    x_pad = jnp.pad(x, ((0, 0), (1, 1), (1, 1)))
    x_flat = x_pad.reshape(C_IN, NFLAT)
    x_ext = jnp.pad(x_flat, ((0, 0), (53, XEXT - NFLAT - 53)))

    wconv = jnp.transpose(conv_w, (2, 3, 0, 1)).reshape(9, C_IN, C_IN)
    bconv = conv_b.reshape(C_IN, 1)

    wcls = cls_w.reshape(NUM_ANCHORS, C_IN)
    wreg = reg_w.reshape(NUM_ANCHORS, 4, C_IN).transpose(1, 0, 2).reshape(36, C_IN)
    whead = jnp.concatenate([wcls, wreg, jnp.zeros((3, C_IN), f32)], axis=0)
    bcls = cls_b.reshape(NUM_ANCHORS)
    breg = reg_b.reshape(NUM_ANCHORS, 4).transpose(1, 0).reshape(36)
    bhead = jnp.concatenate([bcls, breg, jnp.zeros((3,), f32)]).reshape(48, 1)

    img = jnp.asarray(image_shape, f32).reshape(1, 1)

    shp = (NUM_ANCHORS, NPAD)
    dense_out = pl.pallas_call(
        _dense_body,
        out_shape=[jax.ShapeDtypeStruct(shp, f32)] * 6,
        in_specs=[
            pl.BlockSpec(memory_space=pltpu.VMEM),
            pl.BlockSpec(memory_space=pltpu.VMEM),
            pl.BlockSpec(memory_space=pltpu.VMEM),
            pl.BlockSpec(memory_space=pltpu.VMEM),
            pl.BlockSpec(memory_space=pltpu.VMEM),
            pl.BlockSpec(memory_space=pltpu.VMEM),
            pl.BlockSpec(memory_space=pltpu.VMEM),
            pl.BlockSpec(memory_space=pltpu.VMEM),
            pl.BlockSpec(memory_space=pltpu.VMEM),
            pl.BlockSpec(memory_space=pltpu.VMEM),
            pl.BlockSpec(memory_space=pltpu.SMEM),
        ],
        out_specs=[pl.BlockSpec(memory_space=pltpu.VMEM)] * 6,
    )(x_ext, wconv, bconv, whead, bhead,
      jnp.asarray(_CW), jnp.asarray(_CH), jnp.asarray(_CX), jnp.asarray(_CY),
      jnp.asarray(_POSMASK), img)

    sraw, snms, x1, y1, x2, y2 = dense_out

    def to_sel(a, fill):
        flat = a.reshape(-1)
        return jnp.pad(flat, (0, NSEL - flat.shape[0]),
                       constant_values=fill).reshape(SROWS, 128)

    sraw_s = to_sel(sraw, NEG_INF)
    snms_s = to_sel(snms, NEG_INF)
    x1_s = to_sel(x1, 0.0)
    y1_s = to_sel(y1, 0.0)
    x2_s = to_sel(x2, 0.0)
    y2_s = to_sel(y2, 0.0)

    refidx_s = jnp.pad(jnp.asarray(_REFIDX).reshape(-1),
                       (0, NSEL - NUM_ANCHORS * NPAD),
                       constant_values=BIG_I32).reshape(SROWS, 128)
    ehot = jnp.zeros((8, 128), f32).at[jnp.arange(4), jnp.arange(4)].set(1.0)

    s0_s, fbrow = pl.pallas_call(
        _sel_body,
        out_shape=[jax.ShapeDtypeStruct((SROWS, 128), f32),
                   jax.ShapeDtypeStruct((8, 128), f32)],
        in_specs=[pl.BlockSpec(memory_space=pltpu.VMEM)] * 8,
        out_specs=[pl.BlockSpec(memory_space=pltpu.VMEM)] * 2,
    )(sraw_s, snms_s, x1_s, y1_s, x2_s, y2_s, refidx_s, ehot)

    cs, cx1, cy1, cx2, cy2 = _sc_compact(
        s0_s.reshape(-1), x1_s.reshape(-1), y1_s.reshape(-1),
        x2_s.reshape(-1), y2_s.reshape(-1))

    def to_cmp(a):
        return a[:NCMP].reshape(NCMP // 128, 128)

    out = pl.pallas_call(
        _nms_body,
        out_shape=jax.ShapeDtypeStruct((1024, 128), f32),
        in_specs=[pl.BlockSpec(memory_space=pltpu.VMEM)] * 7,
        out_specs=pl.BlockSpec(memory_space=pltpu.VMEM),
    )(to_cmp(cs), to_cmp(cx1), to_cmp(cy1), to_cmp(cx2), to_cmp(cy2),
      fbrow, ehot)

    return out[:POST_NMS_TOP_N, :4]


# speedup vs baseline: 23.3004x; 3.9477x over previous
import functools
import math

import jax
import jax.numpy as jnp
import numpy as np
from jax import lax
from jax.experimental import pallas as pl
from jax.experimental.pallas import tpu as pltpu
from jax.experimental.pallas import tpu_sc as plsc

C_IN = 256
H = 50
W = 50
NUM_ANCHORS = 9
STRIDE = 16
PRE_NMS_TOP_N = 6000
POST_NMS_TOP_N = 1000
NMS_THRESH = 0.7
MIN_SIZE = 1.0
BBOX_XFORM_CLIP = math.log(1000.0 / 16)

GRID = 52
NFLAT = GRID * GRID
NPAD = 2816
XEXT = 2944
NSEL = 25600
SROWS = 200
NEG_INF = float("-inf")
BIG_I32 = np.int32(1 << 30)


def _build_consts():
    sizes = np.array([128.0, 256.0, 512.0])
    ratios = np.array([0.5, 1.0, 2.0])
    hs, ws = [], []
    for s in sizes:
        for r in ratios:
            hs.append(s * np.sqrt(r))
            ws.append(s / np.sqrt(r))
    hs = np.array(hs, np.float64)
    ws = np.array(ws, np.float64)

    hh = np.arange(GRID)[:, None].repeat(GRID, 1)
    ww = np.arange(GRID)[None, :].repeat(GRID, 0)
    valid = (hh >= 1) & (hh <= H) & (ww >= 1) & (ww <= W)
    h = hh - 1
    w = ww - 1
    cx = (w + 0.5) * STRIDE
    cy = (h + 0.5) * STRIDE

    def flat_pad(a2d, fill):
        flat = a2d.reshape(-1)
        out = np.full((NPAD,), fill, a2d.dtype)
        out[:NFLAT] = flat
        return out

    cxf = flat_pad(cx.astype(np.float32), 0.0)
    cyf = flat_pad(cy.astype(np.float32), 0.0)
    validf = flat_pad(valid, False)

    CW = np.broadcast_to(ws.astype(np.float32)[:, None], (NUM_ANCHORS, NPAD)).copy()
    CH = np.broadcast_to(hs.astype(np.float32)[:, None], (NUM_ANCHORS, NPAD)).copy()
    CX = np.broadcast_to(cxf[None, :], (NUM_ANCHORS, NPAD)).copy()
    CY = np.broadcast_to(cyf[None, :], (NUM_ANCHORS, NPAD)).copy()

    posmask = np.where(validf[None, :], 0.0, NEG_INF).astype(np.float32)
    posmask = np.broadcast_to(posmask, (NUM_ANCHORS, NPAD)).copy()

    hwf = flat_pad((np.minimum(h, H - 1) * W + np.minimum(w, W - 1)).astype(np.int64), 0)
    refidx = hwf[None, :] * NUM_ANCHORS + np.arange(NUM_ANCHORS)[:, None]
    refidx = np.where(np.broadcast_to(validf[None, :], refidx.shape), refidx, BIG_I32)
    refidx = refidx.astype(np.int32)
    return CW, CH, CX, CY, posmask, refidx


_CW, _CH, _CX, _CY, _POSMASK, _REFIDX = _build_consts()
_TAP_OFFS = tuple(kh * GRID + kw for kh in range(3) for kw in range(3))


def _dense_body(x_ext_ref, wconv_ref, bconv_ref, whead_ref, bhead_ref,
                cw_ref, ch_ref, cx_ref, cy_ref, posmask_ref, img_ref,
                sraw_ref, snms_ref, x1_ref, y1_ref, x2_ref, y2_ref):
    acc = jnp.zeros((C_IN, NPAD), jnp.float32)
    for k, off in enumerate(_TAP_OFFS):
        acc += jnp.dot(wconv_ref[k], x_ext_ref[:, off:off + NPAD],
                       preferred_element_type=jnp.float32)
    act = jnp.maximum(acc + bconv_ref[:, 0:1], 0.0)
    heads = jnp.dot(whead_ref[...], act, preferred_element_type=jnp.float32)
    heads = heads + bhead_ref[:, 0:1]

    logits = heads[0:9]
    dx = heads[9:18]
    dy = heads[18:27]
    dw = jnp.minimum(heads[27:36], BBOX_XFORM_CLIP)
    dh = jnp.minimum(heads[36:45], BBOX_XFORM_CLIP)

    cw = cw_ref[...]
    ch = ch_ref[...]
    pcx = dx * cw + cx_ref[...]
    pcy = dy * ch + cy_ref[...]
    pw = jnp.exp(dw) * cw
    ph = jnp.exp(dh) * ch

    img = img_ref[0, 0]
    x1 = jnp.clip(pcx - 0.5 * pw, 0.0, img)
    y1 = jnp.clip(pcy - 0.5 * ph, 0.0, img)
    x2 = jnp.clip(pcx + 0.5 * pw, 0.0, img)
    y2 = jnp.clip(pcy + 0.5 * ph, 0.0, img)

    sraw = logits + posmask_ref[...]
    valid = ((x2 - x1) >= MIN_SIZE) & ((y2 - y1) >= MIN_SIZE)
    snms = jnp.where(valid, sraw, NEG_INF)

    sraw_ref[...] = sraw
    snms_ref[...] = snms
    x1_ref[...] = x1
    y1_ref[...] = y1
    x2_ref[...] = x2
    y2_ref[...] = y2


def _count_ge(keys, cand):
    return jnp.sum((keys >= cand).astype(jnp.int32))


def _sel_body(sraw_ref, snms_ref, x1_ref, y1_ref, x2_ref, y2_ref,
              refidx_ref, ehot_ref, s0_ref, fb_ref):
    sraw = sraw_ref[...]
    bits = jax.lax.bitcast_convert_type(sraw, jnp.int32)
    keys = jnp.where(bits < 0, bits ^ jnp.int32(0x7FFFFFFF), bits)

    cpos = _count_ge(keys, jnp.int32(0))
    k_val = jnp.where(cpos >= PRE_NMS_TOP_N, jnp.int32(0), jnp.int32(-2147483648))
    for bit in range(30, -1, -1):
        cand = k_val | jnp.int32(1 << bit)
        k_val = jnp.where(_count_ge(keys, cand) >= PRE_NMS_TOP_N, cand, k_val)

    c_gt = jnp.sum((keys > k_val).astype(jnp.int32))
    m_ties = PRE_NMS_TOP_N - c_gt
    tie = keys == k_val
    refidx = refidx_ref[...]
    lo = jnp.int32(0)
    hi = jnp.int32((1 << 15) - 1)
    for _ in range(15):
        mid = (lo + hi) // 2
        cnt = jnp.sum((tie & (refidx <= mid)).astype(jnp.int32))
        take = cnt >= m_ties
        hi = jnp.where(take, mid, hi)
        lo = jnp.where(take, lo, mid + 1)
    in_topk = (keys > k_val) | (tie & (refidx <= hi))

    s0_ref[...] = jnp.where(in_topk, snms_ref[...], NEG_INF)

    m0 = jnp.max(sraw)
    i0 = jnp.min(jnp.where(sraw == m0, refidx, BIG_I32))
    ch0 = ((sraw == m0) & (refidx == i0)).astype(jnp.float32)
    fb0 = jnp.sum(ch0 * x1_ref[...])
    fb1 = jnp.sum(ch0 * y1_ref[...])
    fb2 = jnp.sum(ch0 * x2_ref[...])
    fb3 = jnp.sum(ch0 * y2_ref[...])
    fb_ref[0:1, :] = (fb0 * ehot_ref[0:1, :] + fb1 * ehot_ref[1:2, :]
                      + fb2 * ehot_ref[2:3, :] + fb3 * ehot_ref[3:4, :])


SC_NT = 16
SC_CHUNK = 1600
SC_NBLK = SC_CHUNK // 16
SC_PAD = 1664
NCMP = 6272
SC_SEG = NCMP // SC_NT
SC_OUT = NCMP
SC_LS = 1616


def _sc_compact_body(s0_hbm, x1_hbm, y1_hbm, x2_hbm, y2_hbm,
                     os_hbm, ox1_hbm, oy1_hbm, ox2_hbm, oy2_hbm, cnt_hbm,
                     sv, xv1, yv1, xv2, yv2,
                     ls_s, ls_x1, ls_y1, ls_x2, ls_y2, idx1d, idx2d,
                     sh_s, sh_x1, sh_y1, sh_x2, sh_y2,
                     cls_v, clc_v, cntb, ctv, tmpf, sem):
    w = lax.axis_index("s")
    base_in = pl.multiple_of(w * SC_CHUNK, 16)

    pltpu.sync_copy(s0_hbm.at[pl.ds(base_in, SC_CHUNK)], sv.at[pl.ds(0, SC_CHUNK)])
    pltpu.sync_copy(x1_hbm.at[pl.ds(base_in, SC_CHUNK)], xv1.at[pl.ds(0, SC_CHUNK)])
    pltpu.sync_copy(y1_hbm.at[pl.ds(base_in, SC_CHUNK)], yv1.at[pl.ds(0, SC_CHUNK)])
    pltpu.sync_copy(x2_hbm.at[pl.ds(base_in, SC_CHUNK)], xv2.at[pl.ds(0, SC_CHUNK)])
    pltpu.sync_copy(y2_hbm.at[pl.ds(base_in, SC_CHUNK)], yv2.at[pl.ds(0, SC_CHUNK)])

    for j in range(24):
        cls_v[pl.ds(j * 16, 16)] = jnp.full((16,), NEG_INF, jnp.float32)
        clc_v[pl.ds(j * 16, 16)] = jnp.zeros((16,), jnp.float32)
    cls_v[pl.ds(SC_SEG - 16, 16)] = jnp.full((16,), NEG_INF, jnp.float32)
    clc_v[pl.ds(SC_SEG - 16, 16)] = jnp.zeros((16,), jnp.float32)
    seg = pl.multiple_of(w * SC_SEG, 8)
    pltpu.sync_copy(cls_v, os_hbm.at[pl.ds(seg, SC_SEG)])
    pltpu.sync_copy(clc_v, ox1_hbm.at[pl.ds(seg, SC_SEG)])
    pltpu.sync_copy(clc_v, oy1_hbm.at[pl.ds(seg, SC_SEG)])
    pltpu.sync_copy(clc_v, ox2_hbm.at[pl.ds(seg, SC_SEG)])
    pltpu.sync_copy(clc_v, oy2_hbm.at[pl.ds(seg, SC_SEG)])

    for j in range(SC_LS // 16):
        ls_s[pl.ds(j * 16, 16)] = jnp.full((16,), NEG_INF, jnp.float32)
        ls_x1[pl.ds(j * 16, 16)] = jnp.zeros((16,), jnp.float32)
        ls_y1[pl.ds(j * 16, 16)] = jnp.zeros((16,), jnp.float32)
        ls_x2[pl.ds(j * 16, 16)] = jnp.zeros((16,), jnp.float32)
        ls_y2[pl.ds(j * 16, 16)] = jnp.zeros((16,), jnp.float32)
    sbase = pl.multiple_of(w * SC_LS, 16)
    pltpu.sync_copy(ls_s, sh_s.at[pl.ds(sbase, SC_LS)])
    pltpu.sync_copy(ls_x1, sh_x1.at[pl.ds(sbase, SC_LS)])
    pltpu.sync_copy(ls_y1, sh_y1.at[pl.ds(sbase, SC_LS)])
    pltpu.sync_copy(ls_x2, sh_x2.at[pl.ds(sbase, SC_LS)])
    pltpu.sync_copy(ls_y2, sh_y2.at[pl.ds(sbase, SC_LS)])

    ones = jnp.full((16,), 1, jnp.int32)
    zeros16 = ones - ones
    ninf = jnp.full((16,), NEG_INF, jnp.float32)
    onef = jnp.full((16,), 1.0, jnp.float32)
    zerof = jnp.full((16,), 0.0, jnp.float32)
    lane = lax.iota(jnp.int32, 16)

    tmpf[pl.ds(0, 16)] = zerof
    tmpf[pl.ds(32, 16)] = zerof

    def prefix16(vf):
        acc = vf
        for st in (1, 2, 4, 8):
            tmpf[pl.ds(16, 16)] = acc
            acc = acc + tmpf[pl.ds(16 - st, 16)]
        return acc

    def splat_last(nondec):
        m = nondec
        for st in (1, 2, 4, 8):
            tmpf[pl.ds(16, 16)] = m
            m = jnp.maximum(m, tmpf[pl.ds(16 + st, 16)])
        return m

    run = jnp.full((16,), 0, jnp.int32)
    for j in range(SC_NBLK):
        v = sv[pl.ds(j * 16, 16)]
        m = v > ninf
        mif = jnp.where(m, onef, zerof)
        pcf = prefix16(mif)
        dump = jnp.full((16,), SC_CHUNK, jnp.int32) + lane
        rank = jnp.where(m, run + pcf.astype(jnp.int32) - ones, dump)
        idx1d[pl.ds(j * 16, 16)] = rank + jnp.full((16,), 1, jnp.int32) * 0
        run = run + splat_last(pcf).astype(jnp.int32)

    for j in range(SC_NBLK, SC_PAD // 16):
        idx1d[pl.ds(j * 16, 16)] = jnp.full((16,), SC_CHUNK, jnp.int32) + lane
    sbv = jnp.full((16,), 1, jnp.int32) * sbase
    for j in range(SC_PAD // 16):
        dv = idx1d[pl.ds(j * 16, 16)] + sbv
        idx2d[j // 8, pl.ds((j % 8) * 16, 16)] = dv
    for data, shref in ((sv, sh_s), (xv1, sh_x1), (yv1, sh_y1),
                        (xv2, sh_x2), (yv2, sh_y2)):
        handles = [
            pltpu.async_copy(data.at[pl.ds(r * 128, 128)],
                             shref.at[idx2d.at[r]], sem)
            for r in range(SC_PAD // 128)
        ]
        for h in handles:
            h.wait()
    pltpu.sync_copy(sh_s.at[pl.ds(sbase, SC_LS)], ls_s)
    pltpu.sync_copy(sh_x1.at[pl.ds(sbase, SC_LS)], ls_x1)
    pltpu.sync_copy(sh_y1.at[pl.ds(sbase, SC_LS)], ls_y1)
    pltpu.sync_copy(sh_x2.at[pl.ds(sbase, SC_LS)], ls_x2)
    pltpu.sync_copy(sh_y2.at[pl.ds(sbase, SC_LS)], ls_y2)

    ctv[...] = run
    pltpu.sync_copy(ctv, cnt_hbm.at[w])
    plsc.subcore_barrier()
    pltpu.sync_copy(cnt_hbm, cntb)
    counts = zeros16
    for j in range(SC_NT):
        jv = jnp.full((16,), j, jnp.int32)
        counts = jnp.where(lane == jv, cntb[j], counts)
    cr = lax.bitwise_and(counts + jnp.full((16,), 15, jnp.int32),
                         jnp.full((16,), -16, jnp.int32))
    wv = jnp.full((16,), w, jnp.int32)
    cmask = jnp.where(lane < wv, cr.astype(jnp.float32), zerof)
    base = splat_last(prefix16(cmask)).astype(jnp.int32)

    nblk16 = lax.bitwise_and(run + jnp.full((16,), 15, jnp.int32),
                             jnp.full((16,), -16, jnp.int32))
    bs = pl.multiple_of(base[0], 16)
    nb = nblk16[0] // 16

    def drain_body(b, carry):
        off = pl.multiple_of(b * 16, 16)
        pltpu.sync_copy(ls_s.at[pl.ds(off, 16)], os_hbm.at[pl.ds(bs + off, 16)])
        pltpu.sync_copy(ls_x1.at[pl.ds(off, 16)], ox1_hbm.at[pl.ds(bs + off, 16)])
        pltpu.sync_copy(ls_y1.at[pl.ds(off, 16)], oy1_hbm.at[pl.ds(bs + off, 16)])
        pltpu.sync_copy(ls_x2.at[pl.ds(off, 16)], ox2_hbm.at[pl.ds(bs + off, 16)])
        pltpu.sync_copy(ls_y2.at[pl.ds(off, 16)], oy2_hbm.at[pl.ds(bs + off, 16)])
        return carry

    lax.fori_loop(0, nb, drain_body, jnp.int32(0))


def _sc_compact(s0_flat, x1f, y1f, x2f, y2f):
    f32 = jnp.float32
    i32 = jnp.int32
    mesh = plsc.VectorSubcoreMesh(core_axis_name="c", subcore_axis_name="s",
                                  num_cores=1)
    out_type = [jax.ShapeDtypeStruct((SC_OUT,), f32)] * 5 + [
        jax.ShapeDtypeStruct((SC_NT, 16), i32)]
    scratch = [
        pltpu.VMEM((SC_PAD,), f32),
        pltpu.VMEM((SC_PAD,), f32),
        pltpu.VMEM((SC_PAD,), f32),
        pltpu.VMEM((SC_PAD,), f32),
        pltpu.VMEM((SC_PAD,), f32),
        pltpu.VMEM((SC_LS,), f32),
        pltpu.VMEM((SC_LS,), f32),
        pltpu.VMEM((SC_LS,), f32),
        pltpu.VMEM((SC_LS,), f32),
        pltpu.VMEM((SC_LS,), f32),
        pltpu.VMEM((SC_PAD,), i32),
        pltpu.VMEM((SC_PAD // 128, 128), i32),
        pltpu.VMEM_SHARED((SC_NT * SC_LS,), f32),
        pltpu.VMEM_SHARED((SC_NT * SC_LS,), f32),
        pltpu.VMEM_SHARED((SC_NT * SC_LS,), f32),
        pltpu.VMEM_SHARED((SC_NT * SC_LS,), f32),
        pltpu.VMEM_SHARED((SC_NT * SC_LS,), f32),
        pltpu.VMEM((SC_SEG,), f32),
        pltpu.VMEM((SC_SEG,), f32),
        pltpu.VMEM((SC_NT, 16), i32),
        pltpu.VMEM((16,), i32),
        pltpu.VMEM((48,), f32),
        pltpu.SemaphoreType.DMA,
    ]
    fn = pl.kernel(_sc_compact_body, mesh=mesh, out_type=out_type,
                   scratch_types=scratch)
    outs = fn(s0_flat, x1f, y1f, x2f, y2f)
    return outs[:5]


def _nms_body(sc_ref, x1_ref, y1_ref, x2_ref, y2_ref, fb_ref, ehot_ref,
              out_ref):
    s0 = sc_ref[...]
    x1 = x1_ref[...]
    y1 = y1_ref[...]
    x2 = x2_ref[...]
    y2 = y2_ref[...]
    areas = (x2 - x1) * (y2 - y1)

    e0 = ehot_ref[0:1, :]
    e1 = ehot_ref[1:2, :]
    e2 = ehot_ref[2:3, :]
    e3 = ehot_ref[3:4, :]
    fbrow = fb_ref[0:1, :]
    fb0 = jnp.sum(fbrow * e0)
    fb1 = jnp.sum(fbrow * e1)
    fb2 = jnp.sum(fbrow * e2)
    fb3 = jnp.sum(fbrow * e3)

    def body(it, s):
        mval = jnp.max(s)
        cf = (s == mval).astype(jnp.float32)
        isfb = mval == NEG_INF
        b0 = jnp.where(isfb, fb0, jnp.sum(cf * x1))
        b1 = jnp.where(isfb, fb1, jnp.sum(cf * y1))
        b2 = jnp.where(isfb, fb2, jnp.sum(cf * x2))
        b3 = jnp.where(isfb, fb3, jnp.sum(cf * y2))
        ar = (b2 - b0) * (b3 - b1)

        xx1 = jnp.maximum(b0, x1)
        yy1 = jnp.maximum(b1, y1)
        xx2 = jnp.minimum(b2, x2)
        yy2 = jnp.minimum(b3, y2)
        inter = jnp.maximum(xx2 - xx1, 0.0) * jnp.maximum(yy2 - yy1, 0.0)
        iou = inter / (ar + areas - inter + 1e-9)
        s = jnp.where(iou > NMS_THRESH, NEG_INF, s)

        out_ref[pl.ds(it, 1), :] = b0 * e0 + b1 * e1 + b2 * e2 + b3 * e3
        return s

    jax.lax.fori_loop(0, POST_NMS_TOP_N, body, s0)


@functools.partial(jax.jit, static_argnames=())
def kernel(features, conv_w, conv_b, cls_w, cls_b, reg_w, reg_b, image_shape):
    f32 = jnp.float32

    x = features[0].astype(f32)
    x_pad = jnp.pad(x, ((0, 0), (1, 1), (1, 1)))
    x_flat = x_pad.reshape(C_IN, NFLAT)
    x_ext = jnp.pad(x_flat, ((0, 0), (53, XEXT - NFLAT - 53)))

    wconv = jnp.transpose(conv_w, (2, 3, 0, 1)).reshape(9, C_IN, C_IN)
    bconv = conv_b.reshape(C_IN, 1)

    wcls = cls_w.reshape(NUM_ANCHORS, C_IN)
    wreg = reg_w.reshape(NUM_ANCHORS, 4, C_IN).transpose(1, 0, 2).reshape(36, C_IN)
    whead = jnp.concatenate([wcls, wreg, jnp.zeros((3, C_IN), f32)], axis=0)
    bcls = cls_b.reshape(NUM_ANCHORS)
    breg = reg_b.reshape(NUM_ANCHORS, 4).transpose(1, 0).reshape(36)
    bhead = jnp.concatenate([bcls, breg, jnp.zeros((3,), f32)]).reshape(48, 1)

    img = jnp.asarray(image_shape, f32).reshape(1, 1)

    shp = (NUM_ANCHORS, NPAD)
    dense_out = pl.pallas_call(
        _dense_body,
        out_shape=[jax.ShapeDtypeStruct(shp, f32)] * 6,
        in_specs=[
            pl.BlockSpec(memory_space=pltpu.VMEM),
            pl.BlockSpec(memory_space=pltpu.VMEM),
            pl.BlockSpec(memory_space=pltpu.VMEM),
            pl.BlockSpec(memory_space=pltpu.VMEM),
            pl.BlockSpec(memory_space=pltpu.VMEM),
            pl.BlockSpec(memory_space=pltpu.VMEM),
            pl.BlockSpec(memory_space=pltpu.VMEM),
            pl.BlockSpec(memory_space=pltpu.VMEM),
            pl.BlockSpec(memory_space=pltpu.VMEM),
            pl.BlockSpec(memory_space=pltpu.VMEM),
            pl.BlockSpec(memory_space=pltpu.SMEM),
        ],
        out_specs=[pl.BlockSpec(memory_space=pltpu.VMEM)] * 6,
    )(x_ext, wconv, bconv, whead, bhead,
      jnp.asarray(_CW), jnp.asarray(_CH), jnp.asarray(_CX), jnp.asarray(_CY),
      jnp.asarray(_POSMASK), img)

    sraw, snms, x1, y1, x2, y2 = dense_out

    def to_sel(a, fill):
        flat = a.reshape(-1)
        return jnp.pad(flat, (0, NSEL - flat.shape[0]),
                       constant_values=fill).reshape(SROWS, 128)

    sraw_s = to_sel(sraw, NEG_INF)
    snms_s = to_sel(snms, NEG_INF)
    x1_s = to_sel(x1, 0.0)
    y1_s = to_sel(y1, 0.0)
    x2_s = to_sel(x2, 0.0)
    y2_s = to_sel(y2, 0.0)

    refidx_s = jnp.pad(jnp.asarray(_REFIDX).reshape(-1),
                       (0, NSEL - NUM_ANCHORS * NPAD),
                       constant_values=BIG_I32).reshape(SROWS, 128)
    ehot = jnp.zeros((8, 128), f32).at[jnp.arange(4), jnp.arange(4)].set(1.0)

    s0_s, fbrow = pl.pallas_call(
        _sel_body,
        out_shape=[jax.ShapeDtypeStruct((SROWS, 128), f32),
                   jax.ShapeDtypeStruct((8, 128), f32)],
        in_specs=[pl.BlockSpec(memory_space=pltpu.VMEM)] * 8,
        out_specs=[pl.BlockSpec(memory_space=pltpu.VMEM)] * 2,
    )(sraw_s, snms_s, x1_s, y1_s, x2_s, y2_s, refidx_s, ehot)

    cs, cx1, cy1, cx2, cy2 = _sc_compact(
        s0_s.reshape(-1), x1_s.reshape(-1), y1_s.reshape(-1),
        x2_s.reshape(-1), y2_s.reshape(-1))

    def to_cmp(a):
        return a[:NCMP].reshape(NCMP // 128, 128)

    out = pl.pallas_call(
        _nms_body,
        out_shape=jax.ShapeDtypeStruct((1024, 128), f32),
        in_specs=[pl.BlockSpec(memory_space=pltpu.VMEM)] * 7,
        out_specs=pl.BlockSpec(memory_space=pltpu.VMEM),
    )(to_cmp(cs), to_cmp(cx1), to_cmp(cy1), to_cmp(cx2), to_cmp(cy2),
      fbrow, ehot)

    return out[:POST_NMS_TOP_N, :4]
